# baseline (speedup 1.0000x reference)
import functools

import jax
import jax.numpy as jnp
from jax import lax
from jax.experimental import pallas as pl
from jax.experimental.pallas import tpu as pltpu
from jax.experimental.pallas import tpu_sc as plsc

_B, _H, _W, _C = 8, 512, 512, 3
_N = _H * _W
_PLB = _N // 8
_NW = 32
_UNITS = _B * 64
_UPW = _UNITS // _NW
_CH = 512
_NCH = 4096 // _CH
_G = _CH // 16
_D = 3
_U = 4


def _sc_sample(idxv, table):
    mesh = plsc.VectorSubcoreMesh(core_axis_name="c", subcore_axis_name="s")

    @functools.partial(
        pl.kernel,
        out_type=jax.ShapeDtypeStruct((_B * _C * _N,), jnp.float32),
        mesh=mesh,
        compiler_params=pltpu.CompilerParams(
            use_tc_tiling_on_sc=False, needs_layout_passes=False
        ),
        scratch_types=[
            pltpu.VMEM((8192,), jnp.float32),
        ] + [pltpu.VMEM((_CH,), jnp.int32) for _ in range(3 * _D)]
          + [pltpu.VMEM((_CH, 8), jnp.float32) for _ in range(3 * _D)]
          + [
            pltpu.VMEM((4096,), jnp.float32),
            pltpu.VMEM((4096,), jnp.float32),
            pltpu.VMEM((4096,), jnp.float32),
        ] + [pltpu.SemaphoreType.DMA for _ in range(3 * _D + 3)],
    )
    def k(idx_hbm, tab_hbm, out_hbm, coords, *rest):
        ls = rest[0:_D]
        exos = rest[_D:2 * _D]
        orvs = rest[2 * _D:3 * _D]
        gs = rest[3 * _D:6 * _D]
        s0, s1, s2 = rest[6 * _D:6 * _D + 3]
        dsems = rest[6 * _D + 3:9 * _D + 3]
        so0, so1, so2 = rest[9 * _D + 3:9 * _D + 6]
        cid = lax.axis_index("c")
        sid = lax.axis_index("s")
        wid = sid * 2 + cid
        iota = lax.iota(jnp.int32, 16)
        sets = tuple(
            (ls[j], exos[j], orvs[j],
             tuple(gs[3 * j:3 * j + 3]), tuple(dsems[3 * j:3 * j + 3]))
            for j in range(_D))

        def unit_body(uu, _):
            u = wid * _UPW + uu
            b = u >> 6
            hb = u & 63
            ob = (b * _C * 64 + hb) * 4096
            pltpu.sync_copy(idx_hbm.at[pl.ds(b * (2 * _N) + hb * 8192, 8192)],
                            coords)

            @pl.when(uu != 0)
            def _():
                for s, so in ((s0, so0), (s1, so1), (s2, so2)):
                    pltpu.make_async_copy(
                        s, out_hbm.at[pl.ds(ob, 4096)], so).wait()

            def pass1(ch, st):
                l, exo, orv = st[0], st[1], st[2]

                def body(i, _):
                    for j in range(_U):
                        g16 = i * _U + j
                        p0 = g16 * 16 + ch * _CH
                        roff = (p0 >> 7) * 256 + (p0 & 127)
                        sl = pl.ds(g16 * 16, 16)
                        r = coords[pl.ds(roff, 16)]
                        c = coords[pl.ds(roff + 128, 16)]
                        ir = (r + 0.5).astype(jnp.int32)
                        ic = (c + 0.5).astype(jnp.int32)
                        orv[sl] = ir | ic
                        irc = jnp.minimum(jnp.maximum(ir, 0), _H - 1)
                        icc = jnp.minimum(jnp.maximum(ic, 0), _W - 1)
                        l[sl] = (((irc >> 3) << 9) + ((icc >> 7) << 7)
                                 + ((irc & 7) << 4) + ((icc >> 3) & 15))
                        exo[sl] = icc & 7
                    return 0

                lax.fori_loop(0, _G // _U, body, 0)

            def fire(st):
                l, g, d = st[0], st[3], st[4]
                return [
                    pltpu.async_copy(
                        tab_hbm.at[pl.ds((b * _C + c) * _PLB, _PLB)].at[l],
                        g[c], d[c])
                    for c in range(_C)
                ]

            def pass2(ch, st):
                exo, orv, g = st[1], st[2], st[3]

                def body(i, _):
                    for j in range(_U):
                        g16 = i * _U + j
                        sl = pl.ds(g16 * 16, 16)
                        rows = iota + g16 * 16
                        cols = exo[sl]
                        ov = orv[sl]
                        valid = (ov >= 0) & (ov < _H)
                        p0 = g16 * 16 + ch * _CH
                        wo = p0 & 511
                        tpos = (((wo >> 7) << 10) + (((p0 >> 9) & 7) << 7)
                                + (wo & 127))
                        osl = pl.ds(tpos, 16)
                        v0 = plsc.load_gather(g[0], [rows, cols])
                        s0[osl] = jnp.where(valid, v0, 0.0)
                        v1 = plsc.load_gather(g[1], [rows, cols])
                        s1[osl] = jnp.where(valid, v1, 0.0)
                        v2 = plsc.load_gather(g[2], [rows, cols])
                        s2[osl] = jnp.where(valid, v2, 0.0)
                    return 0

                lax.fori_loop(0, _G // _U, body, 0)

            cps = {}
            for ch in range(_NCH):
                st = sets[ch % _D]
                pass1(ch, st)
                cps[ch] = fire(st)
                w = ch - (_D - 1)
                if w >= 0:
                    for cp in cps[w]:
                        cp.wait()
                    pass2(w, sets[w % _D])
            for w in range(max(0, _NCH - _D + 1), _NCH):
                for cp in cps[w]:
                    cp.wait()
                pass2(w, sets[w % _D])

            pltpu.async_copy(s0, out_hbm.at[pl.ds(ob, 4096)], so0)
            pltpu.async_copy(s1, out_hbm.at[pl.ds(ob + 64 * 4096, 4096)], so1)
            pltpu.async_copy(s2, out_hbm.at[pl.ds(ob + 2 * 64 * 4096, 4096)],
                             so2)
            return 0

        lax.fori_loop(0, _UPW, unit_body, 0)

        for s, so in ((s0, so0), (s1, so1), (s2, so2)):
            pltpu.make_async_copy(s, out_hbm.at[pl.ds(0, 4096)], so).wait()

    return k(idxv, table)


def kernel(idx, source):
    idxv = (idx.reshape(_B, 2, _N // 128, 128)
            .transpose(0, 2, 1, 3)
            .reshape(_B * 2 * _N))
    srcv = (source.transpose(0, 3, 1, 2)
            .reshape(_B, _C, _H // 8, 8, _W // 128, 128)
            .transpose(0, 1, 2, 4, 3, 5)
            .reshape(_B * _C * _N // 8, 8))
    out = _sc_sample(idxv, srcv)
    return (out.reshape(_B, _C, _H // 8, _W // 128, 8, 128)
            .transpose(0, 1, 2, 4, 3, 5)
            .reshape(_B, _C, _H, _W)
            .transpose(0, 2, 3, 1))


# speedup vs baseline: 1.0040x; 1.0040x over previous
import functools

import jax
import jax.numpy as jnp
from jax import lax
from jax.experimental import pallas as pl
from jax.experimental.pallas import tpu as pltpu
from jax.experimental.pallas import tpu_sc as plsc

_B, _H, _W, _C = 8, 512, 512, 3
_N = _H * _W
_PLB = _N // 8
_NW = 32
_UNITS = _B * 64
_UPW = _UNITS // _NW
_CH = 512
_NCH = 4096 // _CH
_G = _CH // 16
_D = 5
_U = 1


def _sc_sample(idxv, table):
    mesh = plsc.VectorSubcoreMesh(core_axis_name="c", subcore_axis_name="s")

    @functools.partial(
        pl.kernel,
        out_type=jax.ShapeDtypeStruct((_B * _C * _N,), jnp.float32),
        mesh=mesh,
        compiler_params=pltpu.CompilerParams(
            use_tc_tiling_on_sc=False, needs_layout_passes=False
        ),
        scratch_types=[
            pltpu.VMEM((8192,), jnp.float32),
        ] + [pltpu.VMEM((_CH,), jnp.int32) for _ in range(3 * _D)]
          + [pltpu.VMEM((_CH, 8), jnp.float32) for _ in range(3 * _D)]
          + [
            pltpu.VMEM((4096,), jnp.float32),
            pltpu.VMEM((4096,), jnp.float32),
            pltpu.VMEM((4096,), jnp.float32),
        ] + [pltpu.SemaphoreType.DMA for _ in range(3 * _D + 3)],
    )
    def k(idx_hbm, tab_hbm, out_hbm, coords, *rest):
        ls = rest[0:_D]
        exos = rest[_D:2 * _D]
        orvs = rest[2 * _D:3 * _D]
        gs = rest[3 * _D:6 * _D]
        s0, s1, s2 = rest[6 * _D:6 * _D + 3]
        dsems = rest[6 * _D + 3:9 * _D + 3]
        so0, so1, so2 = rest[9 * _D + 3:9 * _D + 6]
        cid = lax.axis_index("c")
        sid = lax.axis_index("s")
        wid = sid * 2 + cid
        iota = lax.iota(jnp.int32, 16)
        sets = tuple(
            (ls[j], exos[j], orvs[j],
             tuple(gs[3 * j:3 * j + 3]), tuple(dsems[3 * j:3 * j + 3]))
            for j in range(_D))

        def unit_body(uu, _):
            u = wid * _UPW + uu
            b = u >> 6
            hb = u & 63
            ob = (b * _C * 64 + hb) * 4096
            pltpu.sync_copy(idx_hbm.at[pl.ds(b * (2 * _N) + hb * 8192, 8192)],
                            coords)

            @pl.when(uu != 0)
            def _():
                for s, so in ((s0, so0), (s1, so1), (s2, so2)):
                    pltpu.make_async_copy(
                        s, out_hbm.at[pl.ds(ob, 4096)], so).wait()

            def pass1(ch, st):
                l, exo, orv = st[0], st[1], st[2]

                def body(i, _):
                    for j in range(_U):
                        g16 = i * _U + j
                        p0 = g16 * 16 + ch * _CH
                        roff = (p0 >> 7) * 256 + (p0 & 127)
                        sl = pl.ds(g16 * 16, 16)
                        r = coords[pl.ds(roff, 16)]
                        c = coords[pl.ds(roff + 128, 16)]
                        ir = (r + 0.5).astype(jnp.int32)
                        ic = (c + 0.5).astype(jnp.int32)
                        orv[sl] = ir | ic
                        irc = jnp.minimum(jnp.maximum(ir, 0), _H - 1)
                        icc = jnp.minimum(jnp.maximum(ic, 0), _W - 1)
                        l[sl] = (((irc >> 3) << 9) + ((icc >> 7) << 7)
                                 + ((irc & 7) << 4) + ((icc >> 3) & 15))
                        exo[sl] = icc & 7
                    return 0

                lax.fori_loop(0, _G // _U, body, 0)

            def fire(st):
                l, g, d = st[0], st[3], st[4]
                return [
                    pltpu.async_copy(
                        tab_hbm.at[pl.ds((b * _C + c) * _PLB, _PLB)].at[l],
                        g[c], d[c])
                    for c in range(_C)
                ]

            def pass2(ch, st):
                exo, orv, g = st[1], st[2], st[3]

                def body(i, _):
                    for j in range(_U):
                        g16 = i * _U + j
                        sl = pl.ds(g16 * 16, 16)
                        rows = iota + g16 * 16
                        cols = exo[sl]
                        ov = orv[sl]
                        valid = (ov >= 0) & (ov < _H)
                        p0 = g16 * 16 + ch * _CH
                        wo = p0 & 511
                        tpos = (((wo >> 7) << 10) + (((p0 >> 9) & 7) << 7)
                                + (wo & 127))
                        osl = pl.ds(tpos, 16)
                        v0 = plsc.load_gather(g[0], [rows, cols])
                        s0[osl] = jnp.where(valid, v0, 0.0)
                        v1 = plsc.load_gather(g[1], [rows, cols])
                        s1[osl] = jnp.where(valid, v1, 0.0)
                        v2 = plsc.load_gather(g[2], [rows, cols])
                        s2[osl] = jnp.where(valid, v2, 0.0)
                    return 0

                lax.fori_loop(0, _G // _U, body, 0)

            cps = {}
            for ch in range(_NCH):
                st = sets[ch % _D]
                pass1(ch, st)
                cps[ch] = fire(st)
                w = ch - (_D - 1)
                if w >= 0:
                    for cp in cps[w]:
                        cp.wait()
                    pass2(w, sets[w % _D])
            for w in range(max(0, _NCH - _D + 1), _NCH):
                for cp in cps[w]:
                    cp.wait()
                pass2(w, sets[w % _D])

            pltpu.async_copy(s0, out_hbm.at[pl.ds(ob, 4096)], so0)
            pltpu.async_copy(s1, out_hbm.at[pl.ds(ob + 64 * 4096, 4096)], so1)
            pltpu.async_copy(s2, out_hbm.at[pl.ds(ob + 2 * 64 * 4096, 4096)],
                             so2)
            return 0

        lax.fori_loop(0, _UPW, unit_body, 0)

        for s, so in ((s0, so0), (s1, so1), (s2, so2)):
            pltpu.make_async_copy(s, out_hbm.at[pl.ds(0, 4096)], so).wait()

    return k(idxv, table)


def kernel(idx, source):
    idxv = (idx.reshape(_B, 2, _N // 128, 128)
            .transpose(0, 2, 1, 3)
            .reshape(_B * 2 * _N))
    srcv = (source.transpose(0, 3, 1, 2)
            .reshape(_B, _C, _H // 8, 8, _W // 128, 128)
            .transpose(0, 1, 2, 4, 3, 5)
            .reshape(_B * _C * _N // 8, 8))
    out = _sc_sample(idxv, srcv)
    return (out.reshape(_B, _C, _H // 8, _W // 128, 8, 128)
            .transpose(0, 1, 2, 4, 3, 5)
            .reshape(_B, _C, _H, _W)
            .transpose(0, 2, 3, 1))


# speedup vs baseline: 1.3729x; 1.3674x over previous
import functools

import jax
import jax.numpy as jnp
from jax import lax
from jax.experimental import pallas as pl
from jax.experimental.pallas import tpu as pltpu
from jax.experimental.pallas import tpu_sc as plsc

_B, _H, _W, _C = 8, 512, 512, 3
_N = _H * _W
_PLB = _N // 8
_NW = 32
_CH = 512
_NCH = 4096 // _CH
_G = _CH // 16
_D = 3
_BPS = _B // 2
_SHB = _C * _PLB


def _sc_sample(idxv, table):
    mesh = plsc.VectorSubcoreMesh(core_axis_name="c", subcore_axis_name="s")

    @functools.partial(
        pl.kernel,
        out_type=jax.ShapeDtypeStruct((_B * _C * _N,), jnp.float32),
        mesh=mesh,
        compiler_params=pltpu.CompilerParams(
            use_tc_tiling_on_sc=False, needs_layout_passes=False
        ),
        scratch_types=[
            pltpu.VMEM((8192,), jnp.float32),
        ] + [pltpu.VMEM((_CH,), jnp.int32) for _ in range(3 * _D)]
          + [pltpu.VMEM((_CH, 8), jnp.float32) for _ in range(3 * _D)]
          + [
            pltpu.VMEM((4096,), jnp.float32),
            pltpu.VMEM((4096,), jnp.float32),
            pltpu.VMEM((4096,), jnp.float32),
            pltpu.VMEM_SHARED((_SHB, 8), jnp.float32),
        ] + [pltpu.SemaphoreType.DMA for _ in range(3 * _D + 4)],
    )
    def k(idx_hbm, tab_hbm, out_hbm, coords, *rest):
        ls = rest[0:_D]
        exos = rest[_D:2 * _D]
        orvs = rest[2 * _D:3 * _D]
        gs = rest[3 * _D:6 * _D]
        s0, s1, s2 = rest[6 * _D:6 * _D + 3]
        shm = rest[6 * _D + 3]
        dsems = rest[6 * _D + 4:9 * _D + 4]
        so0, so1, so2, dstage = rest[9 * _D + 4:9 * _D + 8]
        cid = lax.axis_index("c")
        sid = lax.axis_index("s")
        iota = lax.iota(jnp.int32, 16)
        sets = tuple(
            (ls[j], exos[j], orvs[j],
             tuple(gs[3 * j:3 * j + 3]), tuple(dsems[3 * j:3 * j + 3]))
            for j in range(_D))

        def stage(bsrc, sdst):
            return pltpu.make_async_copy(
                tab_hbm.at[pl.ds(bsrc * _SHB, _SHB)],
                shm.at[pl.ds(sdst * _SHB, _SHB)], dstage)

        def batch_body(bi, _):
            b = cid * _BPS + bi
            sbase = 0

            @pl.when(sid == 0)
            def _():
                cp = stage(b, 0)
                cp.start()
                cp.wait()

            plsc.subcore_barrier()

            def unit_body(uu, _):
                hb = sid * 4 + uu
                ob = (b * _C * 64 + hb) * 4096
                pltpu.sync_copy(
                    idx_hbm.at[pl.ds(b * (2 * _N) + hb * 8192, 8192)], coords)

                @pl.when(jnp.logical_or(bi != 0, uu != 0))
                def _():
                    for s, so in ((s0, so0), (s1, so1), (s2, so2)):
                        pltpu.make_async_copy(
                            s, out_hbm.at[pl.ds(ob, 4096)], so).wait()

                def pass1(ch, st):
                    l, exo, orv = st[0], st[1], st[2]

                    def body(i, _):
                        p0 = i * 16 + ch * _CH
                        roff = (p0 >> 7) * 256 + (p0 & 127)
                        sl = pl.ds(i * 16, 16)
                        r = coords[pl.ds(roff, 16)]
                        c = coords[pl.ds(roff + 128, 16)]
                        ir = (r + 0.5).astype(jnp.int32)
                        ic = (c + 0.5).astype(jnp.int32)
                        orv[sl] = ir | ic
                        irc = jnp.minimum(jnp.maximum(ir, 0), _H - 1)
                        icc = jnp.minimum(jnp.maximum(ic, 0), _W - 1)
                        l[sl] = (((irc >> 3) << 9) + ((icc >> 7) << 7)
                                 + ((irc & 7) << 4) + ((icc >> 3) & 15))
                        exo[sl] = icc & 7
                        return 0

                    lax.fori_loop(0, _G, body, 0)

                def fire(st):
                    l, g, d = st[0], st[3], st[4]
                    return [
                        pltpu.async_copy(
                            shm.at[pl.ds(sbase + c * _PLB, _PLB)].at[l],
                            g[c], d[c])
                        for c in range(_C)
                    ]

                def pass2(ch, st):
                    exo, orv, g = st[1], st[2], st[3]

                    def body(i, _):
                        sl = pl.ds(i * 16, 16)
                        rows = iota + i * 16
                        cols = exo[sl]
                        ov = orv[sl]
                        valid = (ov >= 0) & (ov < _H)
                        p0 = i * 16 + ch * _CH
                        wo = p0 & 511
                        tpos = (((wo >> 7) << 10) + (((p0 >> 9) & 7) << 7)
                                + (wo & 127))
                        osl = pl.ds(tpos, 16)
                        v0 = plsc.load_gather(g[0], [rows, cols])
                        s0[osl] = jnp.where(valid, v0, 0.0)
                        v1 = plsc.load_gather(g[1], [rows, cols])
                        s1[osl] = jnp.where(valid, v1, 0.0)
                        v2 = plsc.load_gather(g[2], [rows, cols])
                        s2[osl] = jnp.where(valid, v2, 0.0)
                        return 0

                    lax.fori_loop(0, _G, body, 0)

                cps = {}
                for ch in range(_NCH):
                    st = sets[ch % _D]
                    pass1(ch, st)
                    cps[ch] = fire(st)
                    w = ch - (_D - 1)
                    if w >= 0:
                        for cp in cps[w]:
                            cp.wait()
                        pass2(w, sets[w % _D])
                for w in range(max(0, _NCH - _D + 1), _NCH):
                    for cp in cps[w]:
                        cp.wait()
                    pass2(w, sets[w % _D])

                pltpu.async_copy(s0, out_hbm.at[pl.ds(ob, 4096)], so0)
                pltpu.async_copy(
                    s1, out_hbm.at[pl.ds(ob + 64 * 4096, 4096)], so1)
                pltpu.async_copy(
                    s2, out_hbm.at[pl.ds(ob + 2 * 64 * 4096, 4096)], so2)
                return 0

            lax.fori_loop(0, 4, unit_body, 0)

            plsc.subcore_barrier()
            return 0

        lax.fori_loop(0, _BPS, batch_body, 0)

        for s, so in ((s0, so0), (s1, so1), (s2, so2)):
            pltpu.make_async_copy(s, out_hbm.at[pl.ds(0, 4096)], so).wait()

    return k(idxv, table)


def kernel(idx, source):
    idxv = (idx.reshape(_B, 2, _N // 128, 128)
            .transpose(0, 2, 1, 3)
            .reshape(_B * 2 * _N))
    srcv = (source.transpose(0, 3, 1, 2)
            .reshape(_B, _C, _H // 8, 8, _W // 128, 128)
            .transpose(0, 1, 2, 4, 3, 5)
            .reshape(_B * _C * _N // 8, 8))
    out = _sc_sample(idxv, srcv)
    return (out.reshape(_B, _C, _H // 8, _W // 128, 8, 128)
            .transpose(0, 1, 2, 4, 3, 5)
            .reshape(_B, _C, _H, _W)
            .transpose(0, 2, 3, 1))
